# Initial kernel scaffold; baseline (speedup 1.0000x reference)
import jax
import jax.numpy as jnp
from jax.experimental import pallas as pl


def kernel(x, edge_index, batch, W1, b1, W2, b2, W3, b3, Wl, bl):
    raise NotImplementedError("write your pallas kernel here")


# speedup vs baseline: 11.8912x; 11.8912x over previous
import functools

import jax
import jax.numpy as jnp
from jax import lax
from jax.experimental import pallas as pl
from jax.experimental.pallas import tpu as pltpu
from jax.experimental.pallas import tpu_sc as plsc

f32 = jnp.float32
i32 = jnp.int32

NC = 2
NS = 16
WIN = 128

NUM_GRAPHS = 1024


def _mesh():
  return plsc.VectorSubcoreMesh(core_axis_name="c", subcore_axis_name="s")


def _round_up(v, m):
  return ((v + m - 1) // m) * m


def _pick_bn(np_rows, cap=8192):
  best = 8
  for k in range(1, np_rows + 1):
    if np_rows % k == 0:
      bn = np_rows // k
      if bn <= cap and bn % 8 == 0:
        return bn
      if bn < 8:
        break
  return best


def _make_deg_counts(WE, WB, NP, GP):
  we_per = WE // (NC * NS)
  wb_per = WB // (NC * NS)
  rows_d = NP // NS
  rows_c = GP // NS

  @functools.partial(
      pl.kernel,
      out_type=[
          jax.ShapeDtypeStruct((NC, NP, 8), f32),
          jax.ShapeDtypeStruct((NC, GP, 8), f32),
      ],
      mesh=_mesh(),
      compiler_params=pltpu.CompilerParams(use_tc_tiling_on_sc=False),
      scratch_types=[
          pltpu.VMEM_SHARED((NP, 8), f32),
          pltpu.VMEM_SHARED((GP, 8), f32),
          pltpu.VMEM((WIN, 8), f32),
          pltpu.VMEM((WIN,), i32),
      ],
  )
  def k(dst_w, batch_w, ones_hbm, zd_hbm, zc_hbm, degp, cntp,
        dacc, cacc, ones_v, idx_v):
    c = lax.axis_index("c")
    s = lax.axis_index("s")
    wid = c * NS + s
    dsl = pl.ds(s * rows_d, rows_d)
    csl = pl.ds(s * rows_c, rows_c)
    pltpu.sync_copy(zd_hbm, dacc.at[dsl, :])
    pltpu.sync_copy(zc_hbm, cacc.at[csl, :])
    pltpu.sync_copy(ones_hbm, ones_v)
    plsc.subcore_barrier()

    @pl.loop(0, we_per)
    def _(j):
      pltpu.sync_copy(dst_w.at[wid * we_per + j], idx_v)
      pltpu.sync_copy(ones_v, dacc.at[idx_v], add=True)

    @pl.loop(0, wb_per)
    def _(j):
      pltpu.sync_copy(batch_w.at[wid * wb_per + j], idx_v)
      pltpu.sync_copy(ones_v, cacc.at[idx_v], add=True)

    plsc.subcore_barrier()
    pltpu.sync_copy(dacc.at[dsl, :], degp.at[c, dsl, :])
    pltpu.sync_copy(cacc.at[csl, :], cntp.at[c, csl, :])

  return k


def _make_agg16(WE, NP):
  we_per = WE // (NC * NS)
  rows = NP // NS

  @functools.partial(
      pl.kernel,
      out_type=[jax.ShapeDtypeStruct((NC, NP, 16), f32)],
      mesh=_mesh(),
      compiler_params=pltpu.CompilerParams(use_tc_tiling_on_sc=False),
      scratch_types=[
          pltpu.VMEM_SHARED((NP, 16), f32),
          pltpu.VMEM((WIN,), i32),
          pltpu.VMEM((WIN,), i32),
          pltpu.VMEM((WIN, 16), f32),
          pltpu.SemaphoreType.DMA,
      ],
  )
  def k(p0, src_w, dst_w, z_hbm, qp, acc, si_v, di_v, rows_v, sem):
    c = lax.axis_index("c")
    s = lax.axis_index("s")
    wid = c * NS + s
    sl = pl.ds(s * rows, rows)
    pltpu.sync_copy(z_hbm, acc.at[sl, :])
    plsc.subcore_barrier()

    @pl.loop(0, we_per)
    def _(j):
      w = wid * we_per + j
      pltpu.sync_copy(src_w.at[w], si_v)
      pltpu.sync_copy(dst_w.at[w], di_v)
      pltpu.async_copy(p0.at[si_v], rows_v, sem).wait()
      pltpu.sync_copy(rows_v, acc.at[di_v], add=True)

    plsc.subcore_barrier()
    pltpu.sync_copy(acc.at[sl, :], qp.at[c, sl, :])

  return k


def _make_agg32(WE, NP):
  w_per = WE // NS
  rows = NP // NS

  @functools.partial(
      pl.kernel,
      out_type=[
          jax.ShapeDtypeStruct((NP, 32), f32),
          jax.ShapeDtypeStruct((NP, 32), f32),
      ],
      mesh=_mesh(),
      compiler_params=pltpu.CompilerParams(use_tc_tiling_on_sc=False),
      scratch_types=[
          pltpu.VMEM_SHARED((NP, 32), f32),
          pltpu.VMEM((WIN,), i32),
          pltpu.VMEM((WIN,), i32),
          pltpu.VMEM((WIN, 32), f32),
          pltpu.SemaphoreType.DMA,
      ],
  )
  def k(pa, pb, src_w, dst_w, z_hbm, qa, qb, acc, si_v, di_v, rows_v, sem):
    c = lax.axis_index("c")
    s = lax.axis_index("s")
    sl = pl.ds(s * rows, rows)
    pltpu.sync_copy(z_hbm, acc.at[sl, :])
    plsc.subcore_barrier()

    @pl.loop(0, w_per)
    def _(j):
      w = s * w_per + j
      pltpu.sync_copy(src_w.at[w], si_v)
      pltpu.sync_copy(dst_w.at[w], di_v)

      @pl.when(c == 0)
      def _():
        pltpu.async_copy(pa.at[si_v], rows_v, sem).wait()

      @pl.when(c == 1)
      def _():
        pltpu.async_copy(pb.at[si_v], rows_v, sem).wait()

      pltpu.sync_copy(rows_v, acc.at[di_v], add=True)

    plsc.subcore_barrier()

    @pl.when(c == 0)
    def _():
      pltpu.sync_copy(acc.at[sl, :], qa.at[sl, :])

    @pl.when(c == 1)
    def _():
      pltpu.sync_copy(acc.at[sl, :], qb.at[sl, :])

  return k


def _make_pool(WB, NB, GP, H):
  wb_per = WB // (NC * NS)
  rows = GP // NS

  @functools.partial(
      pl.kernel,
      out_type=[jax.ShapeDtypeStruct((NC, GP, H), f32)],
      mesh=_mesh(),
      compiler_params=pltpu.CompilerParams(use_tc_tiling_on_sc=False),
      scratch_types=[
          pltpu.VMEM_SHARED((GP, H), f32),
          pltpu.VMEM((WIN,), i32),
          pltpu.VMEM((WIN, H), f32),
      ],
  )
  def k(h3, batch_w, z_hbm, sp, acc, bi_v, rows_v):
    c = lax.axis_index("c")
    s = lax.axis_index("s")
    wid = c * NS + s
    sl = pl.ds(s * rows, rows)
    pltpu.sync_copy(z_hbm, acc.at[sl, :])
    plsc.subcore_barrier()

    @pl.loop(0, wb_per)
    def _(j):
      w = wid * wb_per + j
      pltpu.sync_copy(batch_w.at[w], bi_v)
      pltpu.sync_copy(h3.at[pl.ds(w * WIN, WIN), :], rows_v)
      pltpu.sync_copy(rows_v, acc.at[bi_v], add=True)

    plsc.subcore_barrier()
    pltpu.sync_copy(acc.at[sl, :], sp.at[c, sl, :])

  return k


def _prep_body(degp, x, dinv, p0):
  deg = degp[0, :, 0:1] + degp[1, :, 0:1] + 1.0
  dv = lax.rsqrt(jnp.maximum(deg, 1.0))
  dinv[...] = dv
  xv = x[...] * dv
  pad = jnp.zeros((xv.shape[0], 16 - xv.shape[1]), f32)
  p0[...] = jnp.concatenate([xv, pad], axis=1)


def _layer1_body(q0p, p0, dinv, W1, b1, pa, pb):
  dv = dinv[...]
  agg = (q0p[0] + q0p[1] + p0[...]) * dv
  h = jnp.dot(agg, W1[...], preferred_element_type=f32) + b1[...]
  p = jnp.maximum(h, 0.0) * dv
  pa[...] = p[:, :32]
  pb[...] = p[:, 32:]


def _layer_mid_body(qa, qb, pa, pb, dinv, W, b, oa, ob):
  dv = dinv[...]
  agg = jnp.concatenate([qa[...] + pa[...], qb[...] + pb[...]], axis=1) * dv
  h = jnp.dot(agg, W[...], preferred_element_type=f32) + b[...]
  p = jnp.maximum(h, 0.0) * dv
  oa[...] = p[:, :32]
  ob[...] = p[:, 32:]


def _layer3_body(qa, qb, pa, pb, dinv, W, b, h3):
  dv = dinv[...]
  agg = jnp.concatenate([qa[...] + pa[...], qb[...] + pb[...]], axis=1) * dv
  h3[...] = jnp.dot(agg, W[...], preferred_element_type=f32) + b[...]


def _head_body(sp, cp, Wl, bl, hG, logp):
  s = sp[0] + sp[1]
  cnt = cp[0, :, 0:1] + cp[1, :, 0:1]
  hg = s / jnp.maximum(cnt, 1.0)
  hG[...] = hg
  logits = jnp.dot(hg, Wl[...], preferred_element_type=f32) + bl[...]
  m = jnp.max(logits, axis=1, keepdims=True)
  lse = jnp.log(jnp.sum(jnp.exp(logits - m), axis=1, keepdims=True)) + m
  logp[...] = logits - lse


def _full(block, ndim):
  del ndim
  return pl.BlockSpec(block, lambda i: tuple(0 for _ in block))


def kernel(x, edge_index, batch, W1, b1, W2, b2, W3, b3, Wl, bl):
  N, F = x.shape
  E = edge_index.shape[1]
  H = W1.shape[1]
  C = Wl.shape[1]
  G = NUM_GRAPHS

  WE = _round_up(pl.cdiv(E, WIN), NC * NS)
  Ep = WE * WIN
  NP = _round_up(N + 8, 128)
  WB = _round_up(pl.cdiv(N, WIN), NC * NS)
  NB = WB * WIN
  GP = _round_up(G + 16, 128)

  src = edge_index[0].astype(i32)
  dst = edge_index[1].astype(i32)
  epad = Ep - E
  if epad:
    fill = jnp.arange(epad, dtype=i32)
    src = jnp.concatenate([src, fill % N])
    dst = jnp.concatenate([dst, N + (fill % 8)])
  src_w = src.reshape(WE, WIN)
  dst_w = dst.reshape(WE, WIN)

  bpad = NB - N
  batch_i = batch.astype(i32)
  if bpad:
    fill = jnp.arange(bpad, dtype=i32)
    batch_i = jnp.concatenate([batch_i, G + (fill % 16)])
  batch_w = batch_i.reshape(WB, WIN)

  ones8 = jnp.ones((WIN, 8), f32)
  zd = jnp.zeros((NP // NS, 8), f32)
  zc = jnp.zeros((GP // NS, 8), f32)
  z16 = jnp.zeros((NP // NS, 16), f32)
  z32 = jnp.zeros((NP // NS, 32), f32)
  zg = jnp.zeros((GP // NS, H), f32)

  W1p = jnp.concatenate([W1, jnp.zeros((16 - F, H), f32)], axis=0)
  b1r = b1.reshape(1, H)
  b2r = b2.reshape(1, H)
  b3r = b3.reshape(1, H)
  blr = bl.reshape(1, C)

  degp, cntp = _make_deg_counts(WE, WB, NP, GP)(dst_w, batch_w, ones8, zd, zc)

  BN = _pick_bn(NP)
  grid = (NP // BN,)
  dinv, p0 = pl.pallas_call(
      _prep_body,
      grid=grid,
      in_specs=[
          pl.BlockSpec((2, BN, 8), lambda i: (0, i, 0)),
          pl.BlockSpec((BN, F), lambda i: (i, 0)),
      ],
      out_specs=[
          pl.BlockSpec((BN, 1), lambda i: (i, 0)),
          pl.BlockSpec((BN, 16), lambda i: (i, 0)),
      ],
      out_shape=[
          jax.ShapeDtypeStruct((NP, 1), f32),
          jax.ShapeDtypeStruct((NP, 16), f32),
      ],
  )(degp, x)

  (q0p,) = _make_agg16(WE, NP)(p0, src_w, dst_w, z16)

  p1a, p1b = pl.pallas_call(
      _layer1_body,
      grid=grid,
      in_specs=[
          pl.BlockSpec((2, BN, 16), lambda i: (0, i, 0)),
          pl.BlockSpec((BN, 16), lambda i: (i, 0)),
          pl.BlockSpec((BN, 1), lambda i: (i, 0)),
          _full((16, H), 2),
          _full((1, H), 2),
      ],
      out_specs=[
          pl.BlockSpec((BN, 32), lambda i: (i, 0)),
          pl.BlockSpec((BN, 32), lambda i: (i, 0)),
      ],
      out_shape=[
          jax.ShapeDtypeStruct((NP, 32), f32),
          jax.ShapeDtypeStruct((NP, 32), f32),
      ],
  )(q0p, p0, dinv, W1p, b1r)

  agg32 = _make_agg32(WE, NP)

  def mid_layer(pa, pb, W, b, body, out_specs, out_shape):
    qa, qb = agg32(pa, pb, src_w, dst_w, z32)
    return pl.pallas_call(
        body,
        grid=grid,
        in_specs=[
            pl.BlockSpec((BN, 32), lambda i: (i, 0)),
            pl.BlockSpec((BN, 32), lambda i: (i, 0)),
            pl.BlockSpec((BN, 32), lambda i: (i, 0)),
            pl.BlockSpec((BN, 32), lambda i: (i, 0)),
            pl.BlockSpec((BN, 1), lambda i: (i, 0)),
            _full((H, H), 2),
            _full((1, H), 2),
        ],
        out_specs=out_specs,
        out_shape=out_shape,
    )(qa, qb, pa, pb, dinv, W, b)

  p2a, p2b = mid_layer(
      p1a, p1b, W2, b2r, _layer_mid_body,
      [pl.BlockSpec((BN, 32), lambda i: (i, 0)),
       pl.BlockSpec((BN, 32), lambda i: (i, 0))],
      [jax.ShapeDtypeStruct((NP, 32), f32),
       jax.ShapeDtypeStruct((NP, 32), f32)],
  )

  qa2, qb2 = agg32(p2a, p2b, src_w, dst_w, z32)
  BH = _pick_bn(NB)
  h3 = pl.pallas_call(
      _layer3_body,
      grid=(NB // BH,),
      in_specs=[
          pl.BlockSpec((BH, 32), lambda i: (i, 0)),
          pl.BlockSpec((BH, 32), lambda i: (i, 0)),
          pl.BlockSpec((BH, 32), lambda i: (i, 0)),
          pl.BlockSpec((BH, 32), lambda i: (i, 0)),
          pl.BlockSpec((BH, 1), lambda i: (i, 0)),
          _full((H, H), 2),
          _full((1, H), 2),
      ],
      out_specs=pl.BlockSpec((BH, H), lambda i: (i, 0)),
      out_shape=jax.ShapeDtypeStruct((NB, H), f32),
  )(qa2, qb2, p2a, p2b, dinv, W3, b3r)

  (sp,) = _make_pool(WB, NB, GP, H)(h3, batch_w, zg)

  hG, logp = pl.pallas_call(
      _head_body,
      grid=(1,),
      in_specs=[
          pl.BlockSpec((2, G, H), lambda i: (0, 0, 0)),
          pl.BlockSpec((2, G, 8), lambda i: (0, 0, 0)),
          _full((H, C), 2),
          _full((1, C), 2),
      ],
      out_specs=[
          pl.BlockSpec((G, H), lambda i: (0, 0)),
          pl.BlockSpec((G, C), lambda i: (0, 0)),
      ],
      out_shape=[
          jax.ShapeDtypeStruct((G, H), f32),
          jax.ShapeDtypeStruct((G, C), f32),
      ],
  )(sp, cntp, Wl, blr)

  return (hG, logp)


# speedup vs baseline: 23.2395x; 1.9543x over previous
import functools

import jax
import jax.numpy as jnp
from jax import lax
from jax.experimental import pallas as pl
from jax.experimental.pallas import tpu as pltpu
from jax.experimental.pallas import tpu_sc as plsc

f32 = jnp.float32
i32 = jnp.int32

NC = 2
NS = 16
WIN = 128

NUM_GRAPHS = 1024


def _mesh():
  return plsc.VectorSubcoreMesh(core_axis_name="c", subcore_axis_name="s")


def _round_up(v, m):
  return ((v + m - 1) // m) * m


def _pick_bn(np_rows, cap=8192):
  best = 8
  for k in range(1, np_rows + 1):
    if np_rows % k == 0:
      bn = np_rows // k
      if bn <= cap and bn % 8 == 0:
        return bn
      if bn < 8:
        break
  return best


def _make_deg_counts(WE, WB, NP, GP):
  we_per = WE // (NC * NS)
  wb_per = WB // (NC * NS)
  rows_d = NP // NS
  rows_c = GP // NS

  @functools.partial(
      pl.kernel,
      out_type=[
          jax.ShapeDtypeStruct((NC, NP, 8), f32),
          jax.ShapeDtypeStruct((NC, GP, 8), f32),
      ],
      mesh=_mesh(),
      compiler_params=pltpu.CompilerParams(use_tc_tiling_on_sc=False),
      scratch_types=[
          pltpu.VMEM_SHARED((NP, 8), f32),
          pltpu.VMEM_SHARED((GP, 8), f32),
          pltpu.VMEM((WIN, 8), f32),
          pltpu.VMEM((WIN,), i32),
      ],
  )
  def k(dst_w, batch_w, ones_hbm, zd_hbm, zc_hbm, degp, cntp,
        dacc, cacc, ones_v, idx_v):
    c = lax.axis_index("c")
    s = lax.axis_index("s")
    wid = c * NS + s
    dsl = pl.ds(s * rows_d, rows_d)
    csl = pl.ds(s * rows_c, rows_c)
    pltpu.sync_copy(zd_hbm, dacc.at[dsl, :])
    pltpu.sync_copy(zc_hbm, cacc.at[csl, :])
    pltpu.sync_copy(ones_hbm, ones_v)
    plsc.subcore_barrier()

    @pl.loop(0, we_per)
    def _(j):
      pltpu.sync_copy(dst_w.at[wid * we_per + j], idx_v)
      pltpu.sync_copy(ones_v, dacc.at[idx_v], add=True)

    @pl.loop(0, wb_per)
    def _(j):
      pltpu.sync_copy(batch_w.at[wid * wb_per + j], idx_v)
      pltpu.sync_copy(ones_v, cacc.at[idx_v], add=True)

    plsc.subcore_barrier()
    pltpu.sync_copy(dacc.at[dsl, :], degp.at[c, dsl, :])
    pltpu.sync_copy(cacc.at[csl, :], cntp.at[c, csl, :])

  return k


def _gather_scatter_chunks(src_w, dst_w, gather_fn, wait_fn, acc, si_c, di_c,
                           r0, r1, s0, s1, base, n_ch, ch):
  @pl.loop(0, n_ch)
  def _(cc):
    wb = base + cc * ch
    pltpu.sync_copy(src_w.at[pl.ds(wb, ch), :], si_c)
    pltpu.sync_copy(dst_w.at[pl.ds(wb, ch), :], di_c)
    gather_fn(si_c.at[0], r0, s0)
    for j in range(ch):
      buf, sem = (r0, s0) if j % 2 == 0 else (r1, s1)
      if j + 1 < ch:
        nbuf, nsem = (r1, s1) if j % 2 == 0 else (r0, s0)
        gather_fn(si_c.at[j + 1], nbuf, nsem)
      wait_fn(buf, sem)
      pltpu.sync_copy(buf, acc.at[di_c.at[j]], add=True)


def _make_agg16(WE, NP, CH=14):
  we_per = WE // (NC * NS)
  assert we_per % CH == 0
  rows = NP // NS

  @functools.partial(
      pl.kernel,
      out_type=[jax.ShapeDtypeStruct((NC, NP, 16), f32)],
      mesh=_mesh(),
      compiler_params=pltpu.CompilerParams(use_tc_tiling_on_sc=False),
      scratch_types=[
          pltpu.VMEM_SHARED((NP, 16), f32),
          pltpu.VMEM((CH, WIN), i32),
          pltpu.VMEM((CH, WIN), i32),
          pltpu.VMEM((WIN, 16), f32),
          pltpu.VMEM((WIN, 16), f32),
          pltpu.SemaphoreType.DMA,
          pltpu.SemaphoreType.DMA,
      ],
  )
  def k(p0, src_w, dst_w, z_hbm, qp, acc, si_c, di_c, r0, r1, s0, s1):
    c = lax.axis_index("c")
    s = lax.axis_index("s")
    wid = c * NS + s
    sl = pl.ds(s * rows, rows)
    pltpu.sync_copy(z_hbm, acc.at[sl, :])
    plsc.subcore_barrier()

    def gather(idx, buf, sem):
      pltpu.async_copy(p0.at[idx], buf, sem)

    def wait(buf, sem):
      pltpu.make_async_copy(p0.at[pl.ds(0, WIN), :], buf, sem).wait()

    _gather_scatter_chunks(src_w, dst_w, gather, wait, acc, si_c, di_c,
                           r0, r1, s0, s1, wid * we_per, we_per // CH, CH)

    plsc.subcore_barrier()
    pltpu.sync_copy(acc.at[sl, :], qp.at[c, sl, :])

  return k


def _make_agg32(WE, NP, CH=14):
  w_per = WE // NS
  assert w_per % CH == 0
  rows = NP // NS

  @functools.partial(
      pl.kernel,
      out_type=[
          jax.ShapeDtypeStruct((NP, 32), f32),
          jax.ShapeDtypeStruct((NP, 32), f32),
      ],
      mesh=_mesh(),
      compiler_params=pltpu.CompilerParams(use_tc_tiling_on_sc=False),
      scratch_types=[
          pltpu.VMEM_SHARED((NP, 32), f32),
          pltpu.VMEM((CH, WIN), i32),
          pltpu.VMEM((CH, WIN), i32),
          pltpu.VMEM((WIN, 32), f32),
          pltpu.VMEM((WIN, 32), f32),
          pltpu.SemaphoreType.DMA,
          pltpu.SemaphoreType.DMA,
      ],
  )
  def k(pa, pb, src_w, dst_w, z_hbm, qa, qb, acc, si_c, di_c, r0, r1, s0, s1):
    c = lax.axis_index("c")
    s = lax.axis_index("s")
    sl = pl.ds(s * rows, rows)
    pltpu.sync_copy(z_hbm, acc.at[sl, :])
    plsc.subcore_barrier()

    def gather(idx, buf, sem):
      @pl.when(c == 0)
      def _():
        pltpu.async_copy(pa.at[idx], buf, sem)

      @pl.when(c == 1)
      def _():
        pltpu.async_copy(pb.at[idx], buf, sem)

    def wait(buf, sem):
      pltpu.make_async_copy(pa.at[pl.ds(0, WIN), :], buf, sem).wait()

    _gather_scatter_chunks(src_w, dst_w, gather, wait, acc, si_c, di_c,
                           r0, r1, s0, s1, s * w_per, w_per // CH, CH)

    plsc.subcore_barrier()

    @pl.when(c == 0)
    def _():
      pltpu.sync_copy(acc.at[sl, :], qa.at[sl, :])

    @pl.when(c == 1)
    def _():
      pltpu.sync_copy(acc.at[sl, :], qb.at[sl, :])

  return k


def _make_pool(WB, NB, GP, H):
  wb_per = WB // (NC * NS)
  rows = GP // NS

  @functools.partial(
      pl.kernel,
      out_type=[jax.ShapeDtypeStruct((NC, GP, H), f32)],
      mesh=_mesh(),
      compiler_params=pltpu.CompilerParams(use_tc_tiling_on_sc=False),
      scratch_types=[
          pltpu.VMEM_SHARED((GP, H), f32),
          pltpu.VMEM((WIN,), i32),
          pltpu.VMEM((WIN, H), f32),
      ],
  )
  def k(h3, batch_w, z_hbm, sp, acc, bi_v, rows_v):
    c = lax.axis_index("c")
    s = lax.axis_index("s")
    wid = c * NS + s
    sl = pl.ds(s * rows, rows)
    pltpu.sync_copy(z_hbm, acc.at[sl, :])
    plsc.subcore_barrier()

    @pl.loop(0, wb_per)
    def _(j):
      w = wid * wb_per + j
      pltpu.sync_copy(batch_w.at[w], bi_v)
      pltpu.sync_copy(h3.at[pl.ds(w * WIN, WIN), :], rows_v)
      pltpu.sync_copy(rows_v, acc.at[bi_v], add=True)

    plsc.subcore_barrier()
    pltpu.sync_copy(acc.at[sl, :], sp.at[c, sl, :])

  return k


def _prep_body(degp, x, dinv, p0):
  deg = degp[0, :, 0:1] + degp[1, :, 0:1] + 1.0
  dv = lax.rsqrt(jnp.maximum(deg, 1.0))
  dinv[...] = dv
  xv = x[...] * dv
  pad = jnp.zeros((xv.shape[0], 16 - xv.shape[1]), f32)
  p0[...] = jnp.concatenate([xv, pad], axis=1)


def _layer1_body(q0p, p0, dinv, W1, b1, pa, pb):
  dv = dinv[...]
  agg = (q0p[0] + q0p[1] + p0[...]) * dv
  h = jnp.dot(agg, W1[...], preferred_element_type=f32) + b1[...]
  p = jnp.maximum(h, 0.0) * dv
  pa[...] = p[:, :32]
  pb[...] = p[:, 32:]


def _layer_mid_body(qa, qb, pa, pb, dinv, W, b, oa, ob):
  dv = dinv[...]
  agg = jnp.concatenate([qa[...] + pa[...], qb[...] + pb[...]], axis=1) * dv
  h = jnp.dot(agg, W[...], preferred_element_type=f32) + b[...]
  p = jnp.maximum(h, 0.0) * dv
  oa[...] = p[:, :32]
  ob[...] = p[:, 32:]


def _layer3_body(qa, qb, pa, pb, dinv, W, b, h3):
  dv = dinv[...]
  agg = jnp.concatenate([qa[...] + pa[...], qb[...] + pb[...]], axis=1) * dv
  h3[...] = jnp.dot(agg, W[...], preferred_element_type=f32) + b[...]


def _head_body(sp, cp, Wl, bl, hG, logp):
  s = sp[0] + sp[1]
  cnt = cp[0, :, 0:1] + cp[1, :, 0:1]
  hg = s / jnp.maximum(cnt, 1.0)
  hG[...] = hg
  logits = jnp.dot(hg, Wl[...], preferred_element_type=f32) + bl[...]
  m = jnp.max(logits, axis=1, keepdims=True)
  lse = jnp.log(jnp.sum(jnp.exp(logits - m), axis=1, keepdims=True)) + m
  logp[...] = logits - lse


def _full(block, ndim):
  del ndim
  return pl.BlockSpec(block, lambda i: tuple(0 for _ in block))


def kernel(x, edge_index, batch, W1, b1, W2, b2, W3, b3, Wl, bl):
  N, F = x.shape
  E = edge_index.shape[1]
  H = W1.shape[1]
  C = Wl.shape[1]
  G = NUM_GRAPHS

  WE = _round_up(pl.cdiv(E, WIN), NC * NS)
  Ep = WE * WIN
  NP = _round_up(N + 8, 128)
  WB = _round_up(pl.cdiv(N, WIN), NC * NS)
  NB = WB * WIN
  GP = _round_up(G + 16, 128)

  src = edge_index[0].astype(i32)
  dst = edge_index[1].astype(i32)
  epad = Ep - E
  if epad:
    fill = jnp.arange(epad, dtype=i32)
    src = jnp.concatenate([src, fill % N])
    dst = jnp.concatenate([dst, N + (fill % 8)])
  src_w = src.reshape(WE, WIN)
  dst_w = dst.reshape(WE, WIN)

  bpad = NB - N
  batch_i = batch.astype(i32)
  if bpad:
    fill = jnp.arange(bpad, dtype=i32)
    batch_i = jnp.concatenate([batch_i, G + (fill % 16)])
  batch_w = batch_i.reshape(WB, WIN)

  ones8 = jnp.ones((WIN, 8), f32)
  zd = jnp.zeros((NP // NS, 8), f32)
  zc = jnp.zeros((GP // NS, 8), f32)
  z16 = jnp.zeros((NP // NS, 16), f32)
  z32 = jnp.zeros((NP // NS, 32), f32)
  zg = jnp.zeros((GP // NS, H), f32)

  W1p = jnp.concatenate([W1, jnp.zeros((16 - F, H), f32)], axis=0)
  b1r = b1.reshape(1, H)
  b2r = b2.reshape(1, H)
  b3r = b3.reshape(1, H)
  blr = bl.reshape(1, C)

  degp, cntp = _make_deg_counts(WE, WB, NP, GP)(dst_w, batch_w, ones8, zd, zc)

  BN = _pick_bn(NP)
  grid = (NP // BN,)
  dinv, p0 = pl.pallas_call(
      _prep_body,
      grid=grid,
      in_specs=[
          pl.BlockSpec((2, BN, 8), lambda i: (0, i, 0)),
          pl.BlockSpec((BN, F), lambda i: (i, 0)),
      ],
      out_specs=[
          pl.BlockSpec((BN, 1), lambda i: (i, 0)),
          pl.BlockSpec((BN, 16), lambda i: (i, 0)),
      ],
      out_shape=[
          jax.ShapeDtypeStruct((NP, 1), f32),
          jax.ShapeDtypeStruct((NP, 16), f32),
      ],
  )(degp, x)

  (q0p,) = _make_agg16(WE, NP)(p0, src_w, dst_w, z16)

  p1a, p1b = pl.pallas_call(
      _layer1_body,
      grid=grid,
      in_specs=[
          pl.BlockSpec((2, BN, 16), lambda i: (0, i, 0)),
          pl.BlockSpec((BN, 16), lambda i: (i, 0)),
          pl.BlockSpec((BN, 1), lambda i: (i, 0)),
          _full((16, H), 2),
          _full((1, H), 2),
      ],
      out_specs=[
          pl.BlockSpec((BN, 32), lambda i: (i, 0)),
          pl.BlockSpec((BN, 32), lambda i: (i, 0)),
      ],
      out_shape=[
          jax.ShapeDtypeStruct((NP, 32), f32),
          jax.ShapeDtypeStruct((NP, 32), f32),
      ],
  )(q0p, p0, dinv, W1p, b1r)

  agg32 = _make_agg32(WE, NP)

  def mid_layer(pa, pb, W, b, body, out_specs, out_shape):
    qa, qb = agg32(pa, pb, src_w, dst_w, z32)
    return pl.pallas_call(
        body,
        grid=grid,
        in_specs=[
            pl.BlockSpec((BN, 32), lambda i: (i, 0)),
            pl.BlockSpec((BN, 32), lambda i: (i, 0)),
            pl.BlockSpec((BN, 32), lambda i: (i, 0)),
            pl.BlockSpec((BN, 32), lambda i: (i, 0)),
            pl.BlockSpec((BN, 1), lambda i: (i, 0)),
            _full((H, H), 2),
            _full((1, H), 2),
        ],
        out_specs=out_specs,
        out_shape=out_shape,
    )(qa, qb, pa, pb, dinv, W, b)

  p2a, p2b = mid_layer(
      p1a, p1b, W2, b2r, _layer_mid_body,
      [pl.BlockSpec((BN, 32), lambda i: (i, 0)),
       pl.BlockSpec((BN, 32), lambda i: (i, 0))],
      [jax.ShapeDtypeStruct((NP, 32), f32),
       jax.ShapeDtypeStruct((NP, 32), f32)],
  )

  qa2, qb2 = agg32(p2a, p2b, src_w, dst_w, z32)
  BH = _pick_bn(NB)
  h3 = pl.pallas_call(
      _layer3_body,
      grid=(NB // BH,),
      in_specs=[
          pl.BlockSpec((BH, 32), lambda i: (i, 0)),
          pl.BlockSpec((BH, 32), lambda i: (i, 0)),
          pl.BlockSpec((BH, 32), lambda i: (i, 0)),
          pl.BlockSpec((BH, 32), lambda i: (i, 0)),
          pl.BlockSpec((BH, 1), lambda i: (i, 0)),
          _full((H, H), 2),
          _full((1, H), 2),
      ],
      out_specs=pl.BlockSpec((BH, H), lambda i: (i, 0)),
      out_shape=jax.ShapeDtypeStruct((NB, H), f32),
  )(qa2, qb2, p2a, p2b, dinv, W3, b3r)

  (sp,) = _make_pool(WB, NB, GP, H)(h3, batch_w, zg)

  hG, logp = pl.pallas_call(
      _head_body,
      grid=(1,),
      in_specs=[
          pl.BlockSpec((2, G, H), lambda i: (0, 0, 0)),
          pl.BlockSpec((2, G, 8), lambda i: (0, 0, 0)),
          _full((H, C), 2),
          _full((1, C), 2),
      ],
      out_specs=[
          pl.BlockSpec((G, H), lambda i: (0, 0)),
          pl.BlockSpec((G, C), lambda i: (0, 0)),
      ],
      out_shape=[
          jax.ShapeDtypeStruct((G, H), f32),
          jax.ShapeDtypeStruct((G, C), f32),
      ],
  )(sp, cntp, Wl, blr)

  return (hG, logp)


# speedup vs baseline: 29.0034x; 1.2480x over previous
import functools

import jax
import jax.numpy as jnp
from jax import lax
from jax.experimental import pallas as pl
from jax.experimental.pallas import tpu as pltpu
from jax.experimental.pallas import tpu_sc as plsc

f32 = jnp.float32
i32 = jnp.int32

NC = 2
NS = 16
WIN = 128

NUM_GRAPHS = 1024


def _mesh():
  return plsc.VectorSubcoreMesh(core_axis_name="c", subcore_axis_name="s")


def _round_up(v, m):
  return ((v + m - 1) // m) * m


def _pick_bn(np_rows, cap=8192):
  best = 8
  for k in range(1, np_rows + 1):
    if np_rows % k == 0:
      bn = np_rows // k
      if bn <= cap and bn % 8 == 0:
        return bn
      if bn < 8:
        break
  return best


def _make_deg_counts(WE, WB, NP, GP, CH=14):
  we_per = WE // (NC * NS)
  wb_per = WB // (NC * NS)
  assert we_per % CH == 0
  rows_d = NP // NS
  rows_c = GP // NS

  @functools.partial(
      pl.kernel,
      out_type=[
          jax.ShapeDtypeStruct((NC, NP, 8), f32),
          jax.ShapeDtypeStruct((NC, GP, 8), f32),
      ],
      mesh=_mesh(),
      compiler_params=pltpu.CompilerParams(use_tc_tiling_on_sc=False),
      scratch_types=[
          pltpu.VMEM_SHARED((NP, 8), f32),
          pltpu.VMEM_SHARED((GP, 8), f32),
          pltpu.VMEM((WIN, 8), f32),
          pltpu.VMEM((CH, WIN), i32),
          pltpu.SemaphoreType.DMA,
      ],
  )
  def k(dst_w, batch_w, ones_hbm, zd_hbm, zc_hbm, degp, cntp,
        dacc, cacc, ones_v, idx_c, sem):
    c = lax.axis_index("c")
    s = lax.axis_index("s")
    wid = c * NS + s
    dsl = pl.ds(s * rows_d, rows_d)
    csl = pl.ds(s * rows_c, rows_c)
    pltpu.sync_copy(zd_hbm, dacc.at[dsl, :])
    pltpu.sync_copy(zc_hbm, cacc.at[csl, :])
    pltpu.sync_copy(ones_hbm, ones_v)
    plsc.subcore_barrier()

    def drain(k_copies):
      for _ in range(k_copies):
        pltpu.make_async_copy(ones_hbm, ones_v, sem).wait()

    @pl.loop(0, we_per // CH)
    def _(cc):
      pltpu.sync_copy(dst_w.at[pl.ds(wid * we_per + cc * CH, CH), :], idx_c)
      for j in range(CH):
        pltpu.async_copy(ones_v, dacc.at[idx_c.at[j]], sem, add=True)
      drain(CH)

    n_full = wb_per // CH
    @pl.loop(0, n_full)
    def _(cc):
      pltpu.sync_copy(batch_w.at[pl.ds(wid * wb_per + cc * CH, CH), :], idx_c)
      for j in range(CH):
        pltpu.async_copy(ones_v, cacc.at[idx_c.at[j]], sem, add=True)
      drain(CH)

    rem = wb_per - n_full * CH
    if rem:
      base = wid * wb_per + n_full * CH
      pltpu.sync_copy(batch_w.at[pl.ds(base, rem), :], idx_c.at[pl.ds(0, rem), :])
      for j in range(rem):
        pltpu.async_copy(ones_v, cacc.at[idx_c.at[j]], sem, add=True)
      drain(rem)

    plsc.subcore_barrier()
    pltpu.sync_copy(dacc.at[dsl, :], degp.at[c, dsl, :])
    pltpu.sync_copy(cacc.at[csl, :], cntp.at[c, csl, :])

  return k


def _gather_scatter_chunks(src_w, dst_w, gather_fn, wait_fn, acc, si_c, di_c,
                           bufs, gsems, ssems, base, n_ch, ch):
  nb_ = len(bufs)
  assert nb_ == 4 and ch >= nb_

  @pl.loop(0, n_ch)
  def _(cc):
    wb = base + cc * ch
    pltpu.sync_copy(src_w.at[pl.ds(wb, ch), :], si_c)
    pltpu.sync_copy(dst_w.at[pl.ds(wb, ch), :], di_c)
    gather_fn(si_c.at[0], bufs[0], gsems[0])
    gather_fn(si_c.at[1], bufs[1], gsems[1])
    for j in range(ch):
      b = j % 4
      if j + 2 < ch:
        nb = (j + 2) % 4
        if j >= 2:
          wait_fn(bufs[nb], ssems[nb])
        gather_fn(si_c.at[j + 2], bufs[nb], gsems[nb])
      wait_fn(bufs[b], gsems[b])
      pltpu.async_copy(bufs[b], acc.at[di_c.at[j]], ssems[b], add=True)
    for b in range(4):
      wait_fn(bufs[b], ssems[b])


def _make_agg16(WE, NP, CH=14):
  we_per = WE // (NC * NS)
  assert we_per % CH == 0
  rows = NP // NS

  @functools.partial(
      pl.kernel,
      out_type=[jax.ShapeDtypeStruct((NC, NP, 16), f32)],
      mesh=_mesh(),
      compiler_params=pltpu.CompilerParams(use_tc_tiling_on_sc=False),
      scratch_types=(
          [pltpu.VMEM_SHARED((NP, 16), f32),
           pltpu.VMEM((CH, WIN), i32),
           pltpu.VMEM((CH, WIN), i32)]
          + [pltpu.VMEM((WIN, 16), f32)] * 4
          + [pltpu.SemaphoreType.DMA] * 8
      ),
  )
  def k(p0, src_w, dst_w, z_hbm, qp, acc, si_c, di_c,
        b0, b1, b2, b3, g0, g1, g2, g3, t0, t1, t2, t3):
    c = lax.axis_index("c")
    s = lax.axis_index("s")
    wid = c * NS + s
    sl = pl.ds(s * rows, rows)
    pltpu.sync_copy(z_hbm, acc.at[sl, :])
    plsc.subcore_barrier()

    def gather(idx, buf, sem):
      pltpu.async_copy(p0.at[idx], buf, sem)

    def wait(buf, sem):
      pltpu.make_async_copy(p0.at[pl.ds(0, WIN), :], buf, sem).wait()

    _gather_scatter_chunks(src_w, dst_w, gather, wait, acc, si_c, di_c,
                           [b0, b1, b2, b3], [g0, g1, g2, g3],
                           [t0, t1, t2, t3], wid * we_per, we_per // CH, CH)

    plsc.subcore_barrier()
    pltpu.sync_copy(acc.at[sl, :], qp.at[c, sl, :])

  return k


def _make_agg32(WE, NP, CH=14):
  w_per = WE // NS
  assert w_per % CH == 0
  rows = NP // NS

  @functools.partial(
      pl.kernel,
      out_type=[
          jax.ShapeDtypeStruct((NP, 32), f32),
          jax.ShapeDtypeStruct((NP, 32), f32),
      ],
      mesh=_mesh(),
      compiler_params=pltpu.CompilerParams(use_tc_tiling_on_sc=False),
      scratch_types=(
          [pltpu.VMEM_SHARED((NP, 32), f32),
           pltpu.VMEM((CH, WIN), i32),
           pltpu.VMEM((CH, WIN), i32)]
          + [pltpu.VMEM((WIN, 32), f32)] * 4
          + [pltpu.SemaphoreType.DMA] * 8
      ),
  )
  def k(pa, pb, src_w, dst_w, z_hbm, qa, qb, acc, si_c, di_c,
        b0, b1, b2, b3, g0, g1, g2, g3, t0, t1, t2, t3):
    c = lax.axis_index("c")
    s = lax.axis_index("s")
    sl = pl.ds(s * rows, rows)
    pltpu.sync_copy(z_hbm, acc.at[sl, :])
    plsc.subcore_barrier()

    def gather(idx, buf, sem):
      @pl.when(c == 0)
      def _():
        pltpu.async_copy(pa.at[idx], buf, sem)

      @pl.when(c == 1)
      def _():
        pltpu.async_copy(pb.at[idx], buf, sem)

    def wait(buf, sem):
      pltpu.make_async_copy(pa.at[pl.ds(0, WIN), :], buf, sem).wait()

    _gather_scatter_chunks(src_w, dst_w, gather, wait, acc, si_c, di_c,
                           [b0, b1, b2, b3], [g0, g1, g2, g3],
                           [t0, t1, t2, t3], s * w_per, w_per // CH, CH)

    plsc.subcore_barrier()

    @pl.when(c == 0)
    def _():
      pltpu.sync_copy(acc.at[sl, :], qa.at[sl, :])

    @pl.when(c == 1)
    def _():
      pltpu.sync_copy(acc.at[sl, :], qb.at[sl, :])

  return k


def _make_pool(WB, NB, GP, H):
  wb_per = WB // (NC * NS)
  rows = GP // NS

  @functools.partial(
      pl.kernel,
      out_type=[jax.ShapeDtypeStruct((NC, GP, H), f32)],
      mesh=_mesh(),
      compiler_params=pltpu.CompilerParams(use_tc_tiling_on_sc=False),
      scratch_types=[
          pltpu.VMEM_SHARED((GP, H), f32),
          pltpu.VMEM((WIN,), i32),
          pltpu.VMEM((WIN, H), f32),
      ],
  )
  def k(h3, batch_w, z_hbm, sp, acc, bi_v, rows_v):
    c = lax.axis_index("c")
    s = lax.axis_index("s")
    wid = c * NS + s
    sl = pl.ds(s * rows, rows)
    pltpu.sync_copy(z_hbm, acc.at[sl, :])
    plsc.subcore_barrier()

    @pl.loop(0, wb_per)
    def _(j):
      w = wid * wb_per + j
      pltpu.sync_copy(batch_w.at[w], bi_v)
      pltpu.sync_copy(h3.at[pl.ds(w * WIN, WIN), :], rows_v)
      pltpu.sync_copy(rows_v, acc.at[bi_v], add=True)

    plsc.subcore_barrier()
    pltpu.sync_copy(acc.at[sl, :], sp.at[c, sl, :])

  return k


def _prep_body(degp, x, dinv, p0):
  deg = degp[0, :, 0:1] + degp[1, :, 0:1] + 1.0
  dv = lax.rsqrt(jnp.maximum(deg, 1.0))
  dinv[...] = dv
  xv = x[...] * dv
  pad = jnp.zeros((xv.shape[0], 16 - xv.shape[1]), f32)
  p0[...] = jnp.concatenate([xv, pad], axis=1)


def _layer1_body(q0p, p0, dinv, W1, b1, pa, pb):
  dv = dinv[...]
  agg = (q0p[0] + q0p[1] + p0[...]) * dv
  h = jnp.dot(agg, W1[...], preferred_element_type=f32) + b1[...]
  p = jnp.maximum(h, 0.0) * dv
  pa[...] = p[:, :32]
  pb[...] = p[:, 32:]


def _layer_mid_body(qa, qb, pa, pb, dinv, W, b, oa, ob):
  dv = dinv[...]
  agg = jnp.concatenate([qa[...] + pa[...], qb[...] + pb[...]], axis=1) * dv
  h = jnp.dot(agg, W[...], preferred_element_type=f32) + b[...]
  p = jnp.maximum(h, 0.0) * dv
  oa[...] = p[:, :32]
  ob[...] = p[:, 32:]


def _layer3_body(qa, qb, pa, pb, dinv, W, b, h3):
  dv = dinv[...]
  agg = jnp.concatenate([qa[...] + pa[...], qb[...] + pb[...]], axis=1) * dv
  h3[...] = jnp.dot(agg, W[...], preferred_element_type=f32) + b[...]


def _head_body(sp, cp, Wl, bl, hG, logp):
  s = sp[0] + sp[1]
  cnt = cp[0, :, 0:1] + cp[1, :, 0:1]
  hg = s / jnp.maximum(cnt, 1.0)
  hG[...] = hg
  logits = jnp.dot(hg, Wl[...], preferred_element_type=f32) + bl[...]
  m = jnp.max(logits, axis=1, keepdims=True)
  lse = jnp.log(jnp.sum(jnp.exp(logits - m), axis=1, keepdims=True)) + m
  logp[...] = logits - lse


def _full(block, ndim):
  del ndim
  return pl.BlockSpec(block, lambda i: tuple(0 for _ in block))


def kernel(x, edge_index, batch, W1, b1, W2, b2, W3, b3, Wl, bl):
  N, F = x.shape
  E = edge_index.shape[1]
  H = W1.shape[1]
  C = Wl.shape[1]
  G = NUM_GRAPHS

  WE = _round_up(pl.cdiv(E, WIN), NC * NS)
  Ep = WE * WIN
  NP = _round_up(N + 8, 128)
  WB = _round_up(pl.cdiv(N, WIN), NC * NS)
  NB = WB * WIN
  GP = _round_up(G + 16, 128)

  src = edge_index[0].astype(i32)
  dst = edge_index[1].astype(i32)
  epad = Ep - E
  if epad:
    fill = jnp.arange(epad, dtype=i32)
    src = jnp.concatenate([src, fill % N])
    dst = jnp.concatenate([dst, N + (fill % 8)])
  src_w = src.reshape(WE, WIN)
  dst_w = dst.reshape(WE, WIN)

  bpad = NB - N
  batch_i = batch.astype(i32)
  if bpad:
    fill = jnp.arange(bpad, dtype=i32)
    batch_i = jnp.concatenate([batch_i, G + (fill % 16)])
  batch_w = batch_i.reshape(WB, WIN)

  ones8 = jnp.ones((WIN, 8), f32)
  zd = jnp.zeros((NP // NS, 8), f32)
  zc = jnp.zeros((GP // NS, 8), f32)
  z16 = jnp.zeros((NP // NS, 16), f32)
  z32 = jnp.zeros((NP // NS, 32), f32)
  zg = jnp.zeros((GP // NS, H), f32)

  W1p = jnp.concatenate([W1, jnp.zeros((16 - F, H), f32)], axis=0)
  b1r = b1.reshape(1, H)
  b2r = b2.reshape(1, H)
  b3r = b3.reshape(1, H)
  blr = bl.reshape(1, C)

  degp, cntp = _make_deg_counts(WE, WB, NP, GP)(dst_w, batch_w, ones8, zd, zc)

  BN = _pick_bn(NP)
  grid = (NP // BN,)
  dinv, p0 = pl.pallas_call(
      _prep_body,
      grid=grid,
      in_specs=[
          pl.BlockSpec((2, BN, 8), lambda i: (0, i, 0)),
          pl.BlockSpec((BN, F), lambda i: (i, 0)),
      ],
      out_specs=[
          pl.BlockSpec((BN, 1), lambda i: (i, 0)),
          pl.BlockSpec((BN, 16), lambda i: (i, 0)),
      ],
      out_shape=[
          jax.ShapeDtypeStruct((NP, 1), f32),
          jax.ShapeDtypeStruct((NP, 16), f32),
      ],
  )(degp, x)

  (q0p,) = _make_agg16(WE, NP)(p0, src_w, dst_w, z16)

  p1a, p1b = pl.pallas_call(
      _layer1_body,
      grid=grid,
      in_specs=[
          pl.BlockSpec((2, BN, 16), lambda i: (0, i, 0)),
          pl.BlockSpec((BN, 16), lambda i: (i, 0)),
          pl.BlockSpec((BN, 1), lambda i: (i, 0)),
          _full((16, H), 2),
          _full((1, H), 2),
      ],
      out_specs=[
          pl.BlockSpec((BN, 32), lambda i: (i, 0)),
          pl.BlockSpec((BN, 32), lambda i: (i, 0)),
      ],
      out_shape=[
          jax.ShapeDtypeStruct((NP, 32), f32),
          jax.ShapeDtypeStruct((NP, 32), f32),
      ],
  )(q0p, p0, dinv, W1p, b1r)

  agg32 = _make_agg32(WE, NP)

  def mid_layer(pa, pb, W, b, body, out_specs, out_shape):
    qa, qb = agg32(pa, pb, src_w, dst_w, z32)
    return pl.pallas_call(
        body,
        grid=grid,
        in_specs=[
            pl.BlockSpec((BN, 32), lambda i: (i, 0)),
            pl.BlockSpec((BN, 32), lambda i: (i, 0)),
            pl.BlockSpec((BN, 32), lambda i: (i, 0)),
            pl.BlockSpec((BN, 32), lambda i: (i, 0)),
            pl.BlockSpec((BN, 1), lambda i: (i, 0)),
            _full((H, H), 2),
            _full((1, H), 2),
        ],
        out_specs=out_specs,
        out_shape=out_shape,
    )(qa, qb, pa, pb, dinv, W, b)

  p2a, p2b = mid_layer(
      p1a, p1b, W2, b2r, _layer_mid_body,
      [pl.BlockSpec((BN, 32), lambda i: (i, 0)),
       pl.BlockSpec((BN, 32), lambda i: (i, 0))],
      [jax.ShapeDtypeStruct((NP, 32), f32),
       jax.ShapeDtypeStruct((NP, 32), f32)],
  )

  qa2, qb2 = agg32(p2a, p2b, src_w, dst_w, z32)
  BH = _pick_bn(NB)
  h3 = pl.pallas_call(
      _layer3_body,
      grid=(NB // BH,),
      in_specs=[
          pl.BlockSpec((BH, 32), lambda i: (i, 0)),
          pl.BlockSpec((BH, 32), lambda i: (i, 0)),
          pl.BlockSpec((BH, 32), lambda i: (i, 0)),
          pl.BlockSpec((BH, 32), lambda i: (i, 0)),
          pl.BlockSpec((BH, 1), lambda i: (i, 0)),
          _full((H, H), 2),
          _full((1, H), 2),
      ],
      out_specs=pl.BlockSpec((BH, H), lambda i: (i, 0)),
      out_shape=jax.ShapeDtypeStruct((NB, H), f32),
  )(qa2, qb2, p2a, p2b, dinv, W3, b3r)

  (sp,) = _make_pool(WB, NB, GP, H)(h3, batch_w, zg)

  hG, logp = pl.pallas_call(
      _head_body,
      grid=(1,),
      in_specs=[
          pl.BlockSpec((2, G, H), lambda i: (0, 0, 0)),
          pl.BlockSpec((2, G, 8), lambda i: (0, 0, 0)),
          _full((H, C), 2),
          _full((1, C), 2),
      ],
      out_specs=[
          pl.BlockSpec((G, H), lambda i: (0, 0)),
          pl.BlockSpec((G, C), lambda i: (0, 0)),
      ],
      out_shape=[
          jax.ShapeDtypeStruct((G, H), f32),
          jax.ShapeDtypeStruct((G, C), f32),
      ],
  )(sp, cntp, Wl, blr)

  return (hG, logp)
